# baseline (speedup 1.0000x reference)
import functools

import jax
import jax.numpy as jnp
from jax import lax
from jax.experimental import pallas as pl
from jax.experimental.pallas import tpu as pltpu
from jax.experimental.pallas import tpu_sc as plsc

_NN = 10000
_NE = 320000
_NW = 32
_CH = 128
_NCH = 80
_NG = _CH // 16
_EPW = _CH * _NCH
_NEP = _EPW * _NW
_TABN = 10008
_NNP = 10240
_RPS = _NNP // 16


def _logits(h, a_s, a_d):
    es = jnp.sum(h * a_s, axis=1)
    ed = jnp.sum(h * a_d, axis=1)
    cm = jnp.max(es) + jnp.max(ed)
    c = jnp.where(cm >= 0.0, cm, 0.2 * cm)
    pad = jnp.zeros((_TABN - _NN,), jnp.float32)
    return (jnp.concatenate([es, pad]), jnp.concatenate([ed, pad]),
            jnp.full((1, 16), c, jnp.float32))


def _pad_tab(h):
    d = h.shape[1]
    out = h if d == 8 else jnp.concatenate(
        [h, jnp.zeros((_NN, 8 - d), jnp.float32)], axis=1)
    return jnp.concatenate([out, jnp.zeros((_TABN - _NN, 8), jnp.float32)],
                           axis=0)


def _prep1_body(x_ref, edge_ref, w1_ref, asrc_ref, adst_ref,
                htab_ref, es_ref, ed_ref, c_ref, src3_ref, dst3_ref):
    h = jnp.dot(x_ref[...], w1_ref[...], preferred_element_type=jnp.float32)
    htab_ref[...] = _pad_tab(h)
    es_ref[...], ed_ref[...], c_ref[...] = _logits(
        h, asrc_ref[...], adst_ref[...])
    pads = jnp.full((_NEP - _NE,), _NN, jnp.int32)
    src3_ref[...] = jnp.concatenate(
        [edge_ref[0], pads]).reshape(_NW, _NCH, _CH)
    dst3_ref[...] = jnp.concatenate(
        [edge_ref[1], pads]).reshape(_NW, _NCH, _CH)


_prep1 = pl.pallas_call(
    _prep1_body,
    out_shape=(
        jax.ShapeDtypeStruct((_TABN, 8), jnp.float32),
        jax.ShapeDtypeStruct((_TABN,), jnp.float32),
        jax.ShapeDtypeStruct((_TABN,), jnp.float32),
        jax.ShapeDtypeStruct((1, 16), jnp.float32),
        jax.ShapeDtypeStruct((_NW, _NCH, _CH), jnp.int32),
        jax.ShapeDtypeStruct((_NW, _NCH, _CH), jnp.int32),
    ),
)


def _combine(acc_ref, htab_ref, es_ref, ed_ref, c_ref, d):
    zs = es_ref[: _NN] + ed_ref[: _NN]
    wself = jnp.exp(jnp.maximum(zs, 0.2 * zs) - c_ref[0, 0])
    h = htab_ref[:_NN, :d]
    num = acc_ref[0, :_NN, :d] + acc_ref[1, :_NN, :d] + wself[:, None] * h
    den = acc_ref[0, :_NN, 8] + acc_ref[1, :_NN, 8] + wself
    return num, den


def _mid_body(acc_ref, htab_ref, es_ref, ed_ref, c_ref, b1_ref, w2_ref,
              asrc_ref, adst_ref, htab2_ref, es2_ref, ed2_ref, c2_ref):
    num, den = _combine(acc_ref, htab_ref, es_ref, ed_ref, c_ref, 8)
    h1 = jnp.maximum(num / den[:, None] + b1_ref[...], 0.0)
    h2 = jnp.dot(h1, w2_ref[...], preferred_element_type=jnp.float32)
    htab2_ref[...] = _pad_tab(h2)
    es2_ref[...], ed2_ref[...], c2_ref[...] = _logits(
        h2, asrc_ref[...], adst_ref[...])


_mid = pl.pallas_call(
    _mid_body,
    out_shape=(
        jax.ShapeDtypeStruct((_TABN, 8), jnp.float32),
        jax.ShapeDtypeStruct((_TABN,), jnp.float32),
        jax.ShapeDtypeStruct((_TABN,), jnp.float32),
        jax.ShapeDtypeStruct((1, 16), jnp.float32),
    ),
)


def _final_body(acc_ref, htab_ref, es_ref, ed_ref, c_ref, b2_ref, out_ref):
    num, den = _combine(acc_ref, htab_ref, es_ref, ed_ref, c_ref, 7)
    logits = num / den[:, None] + b2_ref[...]
    m = jnp.max(logits, axis=1, keepdims=True)
    lse = m + jnp.log(jnp.sum(jnp.exp(logits - m), axis=1, keepdims=True))
    out_ref[...] = logits - lse


_final = pl.pallas_call(
    _final_body,
    out_shape=jax.ShapeDtypeStruct((_NN, 7), jnp.float32),
)


_sc_mesh = plsc.VectorSubcoreMesh(core_axis_name="c", subcore_axis_name="s")


def _make_sc_edges(nj):
    @functools.partial(
        pl.kernel,
        out_type=jax.ShapeDtypeStruct((2, _NNP, 16), jnp.float32),
        mesh=_sc_mesh,
        compiler_params=pltpu.CompilerParams(needs_layout_passes=False,
                                             use_tc_tiling_on_sc=False),
        scratch_types=[
            pltpu.VMEM_SHARED((_NNP, 16), jnp.float32),
            pltpu.VMEM((_TABN, 8), jnp.float32),
            pltpu.VMEM((_TABN,), jnp.float32),
            pltpu.VMEM((_TABN,), jnp.float32),
            pltpu.VMEM((16,), jnp.float32),
            [pltpu.VMEM((4, _CH), jnp.int32) for _ in range(2)],
            [pltpu.VMEM((4, _CH), jnp.int32) for _ in range(2)],
            [pltpu.VMEM((_CH, 16), jnp.float32) for _ in range(2)],
            [pltpu.SemaphoreType.DMA for _ in range(2)],
            [pltpu.SemaphoreType.DMA for _ in range(2)],
        ],
    )
    def _sc_edges(src_hbm, dst_hbm, htab_hbm, es_hbm, ed_hbm, c_hbm,
                  zeros_hbm, acc_out, acc_sh, htab_v, es_v, ed_v, c_v,
                  sidxb, didxb, ctrs, sems, isems):
        cid = lax.axis_index("c")
        sid = lax.axis_index("s")
        wid = cid * 16 + sid
        pltpu.sync_copy(htab_hbm, htab_v)
        pltpu.sync_copy(es_hbm, es_v)
        pltpu.sync_copy(ed_hbm, ed_v)
        pltpu.sync_copy(c_hbm.at[0], c_v)
        pltpu.sync_copy(zeros_hbm.at[pl.ds(sid * _RPS, _RPS)],
                        acc_sh.at[pl.ds(sid * _RPS, _RPS)])
        for b in range(2):
            pltpu.sync_copy(zeros_hbm.at[pl.ds(0, _CH)], ctrs[b])

        _NSB = _NCH // 4

        def fire_idx(s, p):
            pltpu.async_copy(src_hbm.at[wid, pl.ds(s * 4, 4)], sidxb[p],
                             isems[p])
            pltpu.async_copy(dst_hbm.at[wid, pl.ds(s * 4, 4)], didxb[p],
                             isems[p])

        def wait_idx(p):
            pltpu.make_async_copy(src_hbm.at[0, pl.ds(0, 4)], sidxb[p],
                                  isems[p]).wait()
            pltpu.make_async_copy(dst_hbm.at[0, pl.ds(0, 4)], didxb[p],
                                  isems[p]).wait()

        fire_idx(0, 0)
        plsc.subcore_barrier()

        lane = lax.iota(jnp.int32, 16)
        rows_g = [lane + (16 * g) for g in range(_NG)]
        col8 = jnp.full((16,), 8, jnp.int32)
        jcols = [jnp.full((16,), j, jnp.int32) for j in range(nj)]
        shift0 = c_v[...]

        def compute_chunk(p, k, buf):
            for g in range(_NG):
                off = g * 16
                s16 = sidxb[p][k, pl.ds(off, 16)]
                d16 = didxb[p][k, pl.ds(off, 16)]
                es = plsc.load_gather(es_v, [s16])
                ed = plsc.load_gather(ed_v, [d16])
                z = es + ed
                w = jnp.exp(jnp.maximum(z, 0.2 * z) - shift0)
                plsc.store_scatter(buf, [rows_g[g], col8], w)
                for j in range(nj):
                    hj = plsc.load_gather(htab_v, [s16, jcols[j]])
                    plsc.store_scatter(buf, [rows_g[g], jcols[j]], w * hj)

        def fire(p, k, b):
            pltpu.async_copy(ctrs[b], acc_sh.at[didxb[p].at[k]], sems[b],
                             add=True)

        def drain(b):
            pltpu.make_async_copy(ctrs[b], acc_sh.at[didxb[0].at[0]],
                                  sems[b]).wait()

        def sb_pair_body(ss, carry):
            for p in range(2):
                s = 2 * ss + p
                wait_idx(p)

                @pl.when(s < _NSB - 1)
                def _():
                    fire_idx(s + 1, 1 - p)

                for k in range(4):
                    b = k % 2

                    @pl.when(s * 4 + k >= 2)
                    def _():
                        drain(b)

                    compute_chunk(p, k, ctrs[b])
                    fire(p, k, b)
            return carry

        lax.fori_loop(0, _NSB // 2, sb_pair_body, 0)
        drain(0)
        drain(1)
        plsc.subcore_barrier()
        pltpu.sync_copy(acc_sh.at[pl.ds(sid * _RPS, _RPS)],
                        acc_out.at[cid, pl.ds(sid * _RPS, _RPS)])

    return _sc_edges


_sc_edges_l1 = _make_sc_edges(8)
_sc_edges_l2 = _make_sc_edges(7)


def kernel(x, edge_index, W1, a_src1, a_dst1, b1, W2, a_src2, a_dst2, b2):
    zeros = jnp.zeros((_NNP, 16), jnp.float32)
    htab1, es1, ed1, c1, src3, dst3 = _prep1(
        x, edge_index.astype(jnp.int32), W1,
        a_src1.reshape(1, 8), a_dst1.reshape(1, 8))
    acc1 = _sc_edges_l1(src3, dst3, htab1, es1, ed1, c1, zeros)
    htab2, es2, ed2, c2 = _mid(acc1, htab1, es1, ed1, c1, b1.reshape(1, 8),
                               W2, a_src2.reshape(1, 7), a_dst2.reshape(1, 7))
    acc2 = _sc_edges_l2(src3, dst3, htab2, es2, ed2, c2, zeros)
    return _final(acc2, htab2, es2, ed2, c2, b2.reshape(1, 7))


# speedup vs baseline: 1.0994x; 1.0994x over previous
import functools

import jax
import jax.numpy as jnp
from jax import lax
from jax.experimental import pallas as pl
from jax.experimental.pallas import tpu as pltpu
from jax.experimental.pallas import tpu_sc as plsc

_NN = 10000
_NE = 320000
_NW = 32
_CH = 128
_NCH = 80
_NG = _CH // 16
_EPW = _CH * _NCH
_NEP = _EPW * _NW
_TABN = 10008
_NNP = 10240
_RPS = _NNP // 16


def _logits(h, a_s, a_d):
    es = jnp.sum(h * a_s, axis=1)
    ed = jnp.sum(h * a_d, axis=1)
    cm = jnp.max(es) + jnp.max(ed)
    c = jnp.where(cm >= 0.0, cm, 0.2 * cm)
    pad = jnp.zeros((_TABN - _NN,), jnp.float32)
    return (jnp.concatenate([es, pad]), jnp.concatenate([ed, pad]),
            jnp.full((1, 16), c, jnp.float32))


def _pad_tab(h):
    d = h.shape[1]
    out = h if d == 8 else jnp.concatenate(
        [h, jnp.zeros((_NN, 8 - d), jnp.float32)], axis=1)
    return jnp.concatenate([out, jnp.zeros((_TABN - _NN, 8), jnp.float32)],
                           axis=0)


def _prep1_body(x_ref, edge_ref, w1_ref, asrc_ref, adst_ref,
                htab_ref, es_ref, ed_ref, c_ref, src3_ref, dst3_ref):
    h = jnp.dot(x_ref[...], w1_ref[...], preferred_element_type=jnp.float32)
    htab_ref[...] = _pad_tab(h)
    es_ref[...], ed_ref[...], c_ref[...] = _logits(
        h, asrc_ref[...], adst_ref[...])
    pads = jnp.full((_NEP - _NE,), _NN, jnp.int32)
    src3_ref[...] = jnp.concatenate(
        [edge_ref[0], pads]).reshape(_NW, _NCH, _CH)
    dst3_ref[...] = jnp.concatenate(
        [edge_ref[1], pads]).reshape(_NW, _NCH, _CH)


_prep1 = pl.pallas_call(
    _prep1_body,
    out_shape=(
        jax.ShapeDtypeStruct((_TABN, 8), jnp.float32),
        jax.ShapeDtypeStruct((_TABN,), jnp.float32),
        jax.ShapeDtypeStruct((_TABN,), jnp.float32),
        jax.ShapeDtypeStruct((1, 16), jnp.float32),
        jax.ShapeDtypeStruct((_NW, _NCH, _CH), jnp.int32),
        jax.ShapeDtypeStruct((_NW, _NCH, _CH), jnp.int32),
    ),
)


def _combine(acc_ref, htab_ref, es_ref, ed_ref, c_ref, d):
    zs = es_ref[: _NN] + ed_ref[: _NN]
    wself = jnp.exp(jnp.maximum(zs, 0.2 * zs) - c_ref[0, 0])
    h = htab_ref[:_NN, :d]
    num = acc_ref[0, :_NN, :d] + acc_ref[1, :_NN, :d] + wself[:, None] * h
    den = acc_ref[0, :_NN, 8] + acc_ref[1, :_NN, 8] + wself
    return num, den


def _mid_body(acc_ref, htab_ref, es_ref, ed_ref, c_ref, b1_ref, w2_ref,
              asrc_ref, adst_ref, htab2_ref, es2_ref, ed2_ref, c2_ref):
    num, den = _combine(acc_ref, htab_ref, es_ref, ed_ref, c_ref, 8)
    h1 = jnp.maximum(num / den[:, None] + b1_ref[...], 0.0)
    h2 = jnp.dot(h1, w2_ref[...], preferred_element_type=jnp.float32)
    htab2_ref[...] = _pad_tab(h2)
    es2_ref[...], ed2_ref[...], c2_ref[...] = _logits(
        h2, asrc_ref[...], adst_ref[...])


_mid = pl.pallas_call(
    _mid_body,
    out_shape=(
        jax.ShapeDtypeStruct((_TABN, 8), jnp.float32),
        jax.ShapeDtypeStruct((_TABN,), jnp.float32),
        jax.ShapeDtypeStruct((_TABN,), jnp.float32),
        jax.ShapeDtypeStruct((1, 16), jnp.float32),
    ),
)


def _final_body(acc_ref, htab_ref, es_ref, ed_ref, c_ref, b2_ref, out_ref):
    num, den = _combine(acc_ref, htab_ref, es_ref, ed_ref, c_ref, 7)
    logits = num / den[:, None] + b2_ref[...]
    m = jnp.max(logits, axis=1, keepdims=True)
    lse = m + jnp.log(jnp.sum(jnp.exp(logits - m), axis=1, keepdims=True))
    out_ref[...] = logits - lse


_final = pl.pallas_call(
    _final_body,
    out_shape=jax.ShapeDtypeStruct((_NN, 7), jnp.float32),
)


_sc_mesh = plsc.VectorSubcoreMesh(core_axis_name="c", subcore_axis_name="s")


def _make_sc_edges(nj):
    @functools.partial(
        pl.kernel,
        out_type=jax.ShapeDtypeStruct((2, _NNP, 16), jnp.float32),
        mesh=_sc_mesh,
        compiler_params=pltpu.CompilerParams(needs_layout_passes=False,
                                             use_tc_tiling_on_sc=False),
        scratch_types=[
            pltpu.VMEM_SHARED((_NNP, 16), jnp.float32),
            pltpu.VMEM((_TABN, 8), jnp.float32),
            pltpu.VMEM((_TABN,), jnp.float32),
            pltpu.VMEM((_TABN,), jnp.float32),
            pltpu.VMEM((16,), jnp.float32),
            [pltpu.VMEM((4, _CH), jnp.int32) for _ in range(2)],
            [pltpu.VMEM((4, _CH), jnp.int32) for _ in range(2)],
            [pltpu.VMEM((_CH, 16), jnp.float32) for _ in range(2)],
            [pltpu.SemaphoreType.DMA for _ in range(2)],
            [pltpu.SemaphoreType.DMA for _ in range(2)],
        ],
    )
    def _sc_edges(src_hbm, dst_hbm, htab_hbm, es_hbm, ed_hbm, c_hbm,
                  zeros_hbm, acc_out, acc_sh, htab_v, es_v, ed_v, c_v,
                  sidxb, didxb, ctrs, sems, isems):
        cid = lax.axis_index("c")
        sid = lax.axis_index("s")
        wid = cid * 16 + sid
        pltpu.sync_copy(htab_hbm, htab_v)
        pltpu.sync_copy(es_hbm, es_v)
        pltpu.sync_copy(ed_hbm, ed_v)
        pltpu.sync_copy(c_hbm.at[0], c_v)
        pltpu.sync_copy(zeros_hbm.at[pl.ds(sid * _RPS, _RPS)],
                        acc_sh.at[pl.ds(sid * _RPS, _RPS)])
        for b in range(2):
            pltpu.sync_copy(zeros_hbm.at[pl.ds(0, _CH)], ctrs[b])

        _NSB = _NCH // 4

        def fire_idx(s, p):
            pltpu.async_copy(src_hbm.at[wid, pl.ds(s * 4, 4)], sidxb[p],
                             isems[p])
            pltpu.async_copy(dst_hbm.at[wid, pl.ds(s * 4, 4)], didxb[p],
                             isems[p])

        def wait_idx(p):
            pltpu.make_async_copy(src_hbm.at[0, pl.ds(0, 4)], sidxb[p],
                                  isems[p]).wait()
            pltpu.make_async_copy(dst_hbm.at[0, pl.ds(0, 4)], didxb[p],
                                  isems[p]).wait()

        fire_idx(0, 0)
        plsc.subcore_barrier()

        lane = lax.iota(jnp.int32, 16)
        rows_g = [lane + (16 * g) for g in range(_NG)]
        col8 = jnp.full((16,), 8, jnp.int32)
        jcols = [jnp.full((16,), j, jnp.int32) for j in range(nj)]
        shift0 = c_v[...]

        def compute_chunk(p, k, buf):
            def group_body(g, carry):
                off = g * 16
                s16 = sidxb[p][k, pl.ds(off, 16)]
                d16 = didxb[p][k, pl.ds(off, 16)]
                es = plsc.load_gather(es_v, [s16])
                ed = plsc.load_gather(ed_v, [d16])
                z = es + ed
                w = jnp.exp(jnp.maximum(z, 0.2 * z) - shift0)
                rows = lane + off
                plsc.store_scatter(buf, [rows, col8], w)
                for j in range(nj):
                    hj = plsc.load_gather(htab_v, [s16, jcols[j]])
                    plsc.store_scatter(buf, [rows, jcols[j]], w * hj)
                return carry

            lax.fori_loop(0, _NG, group_body, 0)

        def fire(p, k, b):
            pltpu.async_copy(ctrs[b], acc_sh.at[didxb[p].at[k]], sems[b],
                             add=True)

        def drain(b):
            pltpu.make_async_copy(ctrs[b], acc_sh.at[didxb[0].at[0]],
                                  sems[b]).wait()

        def sb_pair_body(ss, carry):
            for p in range(2):
                s = 2 * ss + p
                wait_idx(p)

                @pl.when(s < _NSB - 1)
                def _():
                    fire_idx(s + 1, 1 - p)

                for k in range(4):
                    b = k % 2

                    @pl.when(s * 4 + k >= 2)
                    def _():
                        drain(b)

                    compute_chunk(p, k, ctrs[b])
                    fire(p, k, b)
            return carry

        lax.fori_loop(0, _NSB // 2, sb_pair_body, 0)
        drain(0)
        drain(1)
        plsc.subcore_barrier()
        pltpu.sync_copy(acc_sh.at[pl.ds(sid * _RPS, _RPS)],
                        acc_out.at[cid, pl.ds(sid * _RPS, _RPS)])

    return _sc_edges


_sc_edges_l1 = _make_sc_edges(8)
_sc_edges_l2 = _make_sc_edges(7)


def kernel(x, edge_index, W1, a_src1, a_dst1, b1, W2, a_src2, a_dst2, b2):
    zeros = jnp.zeros((_NNP, 16), jnp.float32)
    htab1, es1, ed1, c1, src3, dst3 = _prep1(
        x, edge_index.astype(jnp.int32), W1,
        a_src1.reshape(1, 8), a_dst1.reshape(1, 8))
    acc1 = _sc_edges_l1(src3, dst3, htab1, es1, ed1, c1, zeros)
    htab2, es2, ed2, c2 = _mid(acc1, htab1, es1, ed1, c1, b1.reshape(1, 8),
                               W2, a_src2.reshape(1, 7), a_dst2.reshape(1, 7))
    acc2 = _sc_edges_l2(src3, dst3, htab2, es2, ed2, c2, zeros)
    return _final(acc2, htab2, es2, ed2, c2, b2.reshape(1, 7))


# speedup vs baseline: 1.1231x; 1.0216x over previous
import functools

import jax
import jax.numpy as jnp
from jax import lax
from jax.experimental import pallas as pl
from jax.experimental.pallas import tpu as pltpu
from jax.experimental.pallas import tpu_sc as plsc

_NN = 10000
_NE = 320000
_NW = 32
_CH = 128
_NCH = 81
_NG = _CH // 16
_EPW = _CH * _NCH
_NEP = _EPW * _NW
_TABN = 10008
_NNP = 10240
_RPS = _NNP // 16


def _logits(h, a_s, a_d):
    es = jnp.sum(h * a_s, axis=1)
    ed = jnp.sum(h * a_d, axis=1)
    cm = jnp.max(es) + jnp.max(ed)
    c = jnp.where(cm >= 0.0, cm, 0.2 * cm)
    pad = jnp.zeros((_TABN - _NN,), jnp.float32)
    return (jnp.concatenate([es, pad]), jnp.concatenate([ed, pad]),
            jnp.full((1, 16), c, jnp.float32))


def _pad_tab(h):
    d = h.shape[1]
    out = h if d == 8 else jnp.concatenate(
        [h, jnp.zeros((_NN, 8 - d), jnp.float32)], axis=1)
    return jnp.concatenate([out, jnp.zeros((_TABN - _NN, 8), jnp.float32)],
                           axis=0)


def _prep1_body(x_ref, edge_ref, w1_ref, asrc_ref, adst_ref,
                htab_ref, es_ref, ed_ref, c_ref, src3_ref, dst3_ref):
    h = jnp.dot(x_ref[...], w1_ref[...], preferred_element_type=jnp.float32)
    htab_ref[...] = _pad_tab(h)
    es_ref[...], ed_ref[...], c_ref[...] = _logits(
        h, asrc_ref[...], adst_ref[...])
    loops = lax.iota(jnp.int32, _NN)
    pads = jnp.full((_NEP - _NE - _NN,), _NN, jnp.int32)
    src3_ref[...] = jnp.concatenate(
        [edge_ref[0], loops, pads]).reshape(_NW * _NCH, _CH)
    dst3_ref[...] = jnp.concatenate(
        [edge_ref[1], loops, pads]).reshape(_NW * _NCH, _CH)


_prep1 = pl.pallas_call(
    _prep1_body,
    out_shape=(
        jax.ShapeDtypeStruct((_TABN, 8), jnp.float32),
        jax.ShapeDtypeStruct((_TABN,), jnp.float32),
        jax.ShapeDtypeStruct((_TABN,), jnp.float32),
        jax.ShapeDtypeStruct((1, 16), jnp.float32),
        jax.ShapeDtypeStruct((_NW * _NCH, _CH), jnp.int32),
        jax.ShapeDtypeStruct((_NW * _NCH, _CH), jnp.int32),
    ),
)


def _combine(acc_ref, d):
    num = acc_ref[0, :_NN, :d] + acc_ref[1, :_NN, :d]
    den = acc_ref[0, :_NN, 8] + acc_ref[1, :_NN, 8]
    return num, den


def _mid_body(acc_ref, b1_ref, w2_ref, asrc_ref, adst_ref,
              htab2_ref, es2_ref, ed2_ref, c2_ref):
    num, den = _combine(acc_ref, 8)
    h1 = jnp.maximum(num / den[:, None] + b1_ref[...], 0.0)
    h2 = jnp.dot(h1, w2_ref[...], preferred_element_type=jnp.float32)
    htab2_ref[...] = _pad_tab(h2)
    es2_ref[...], ed2_ref[...], c2_ref[...] = _logits(
        h2, asrc_ref[...], adst_ref[...])


_mid = pl.pallas_call(
    _mid_body,
    out_shape=(
        jax.ShapeDtypeStruct((_TABN, 8), jnp.float32),
        jax.ShapeDtypeStruct((_TABN,), jnp.float32),
        jax.ShapeDtypeStruct((_TABN,), jnp.float32),
        jax.ShapeDtypeStruct((1, 16), jnp.float32),
    ),
)


def _final_body(acc_ref, b2_ref, out_ref):
    num, den = _combine(acc_ref, 7)
    logits = num / den[:, None] + b2_ref[...]
    m = jnp.max(logits, axis=1, keepdims=True)
    lse = m + jnp.log(jnp.sum(jnp.exp(logits - m), axis=1, keepdims=True))
    out_ref[...] = logits - lse


_final = pl.pallas_call(
    _final_body,
    out_shape=jax.ShapeDtypeStruct((_NN, 7), jnp.float32),
)


_sc_mesh = plsc.VectorSubcoreMesh(core_axis_name="c", subcore_axis_name="s")


def _make_sc_edges(nj):
    @functools.partial(
        pl.kernel,
        out_type=jax.ShapeDtypeStruct((2, _NNP, 16), jnp.float32),
        mesh=_sc_mesh,
        compiler_params=pltpu.CompilerParams(needs_layout_passes=False,
                                             use_tc_tiling_on_sc=False),
        scratch_types=[
            pltpu.VMEM_SHARED((_NNP, 16), jnp.float32),
            pltpu.VMEM((_TABN, 8), jnp.float32),
            pltpu.VMEM((_TABN,), jnp.float32),
            pltpu.VMEM((_TABN,), jnp.float32),
            pltpu.VMEM((16,), jnp.float32),
            [pltpu.VMEM((4, _CH), jnp.int32) for _ in range(2)],
            [pltpu.VMEM((4, _CH), jnp.int32) for _ in range(2)],
            [pltpu.VMEM((_CH, 16), jnp.float32) for _ in range(2)],
            [pltpu.SemaphoreType.DMA for _ in range(2)],
            [pltpu.SemaphoreType.DMA for _ in range(2)],
        ],
    )
    def _sc_edges(src_hbm, dst_hbm, htab_hbm, es_hbm, ed_hbm, c_hbm,
                  zeros_hbm, acc_out, acc_sh, htab_v, es_v, ed_v, c_v,
                  sidxb, didxb, ctrs, sems, isems):
        cid = lax.axis_index("c")
        sid = lax.axis_index("s")
        wid = cid * 16 + sid
        pltpu.sync_copy(htab_hbm, htab_v)
        pltpu.sync_copy(es_hbm, es_v)
        pltpu.sync_copy(ed_hbm, ed_v)
        pltpu.sync_copy(c_hbm.at[0], c_v)
        pltpu.sync_copy(zeros_hbm.at[pl.ds(sid * _RPS, _RPS)],
                        acc_sh.at[pl.ds(sid * _RPS, _RPS)])
        for b in range(2):
            pltpu.sync_copy(zeros_hbm.at[pl.ds(0, _CH)], ctrs[b])

        _NSB = (_NCH - 1) // 4

        cbase = wid * _NCH

        def fire_idx(s, p):
            pltpu.async_copy(src_hbm.at[pl.ds(cbase + s * 4, 4)], sidxb[p],
                             isems[p])
            pltpu.async_copy(dst_hbm.at[pl.ds(cbase + s * 4, 4)], didxb[p],
                             isems[p])

        def wait_idx(p):
            pltpu.make_async_copy(src_hbm.at[pl.ds(0, 4)], sidxb[p],
                                  isems[p]).wait()
            pltpu.make_async_copy(dst_hbm.at[pl.ds(0, 4)], didxb[p],
                                  isems[p]).wait()

        fire_idx(0, 0)
        plsc.subcore_barrier()

        lane = lax.iota(jnp.int32, 16)
        rows_g = [lane + (16 * g) for g in range(_NG)]
        col8 = jnp.full((16,), 8, jnp.int32)
        jcols = [jnp.full((16,), j, jnp.int32) for j in range(nj)]
        shift0 = c_v[...]

        def compute_chunk(p, k, buf):
            def group_body(g, carry):
                off = g * 16
                s16 = sidxb[p][k, pl.ds(off, 16)]
                d16 = didxb[p][k, pl.ds(off, 16)]
                es = plsc.load_gather(es_v, [s16])
                ed = plsc.load_gather(ed_v, [d16])
                z = es + ed
                w = jnp.exp(jnp.maximum(z, 0.2 * z) - shift0)
                rows = lane + off
                plsc.store_scatter(buf, [rows, col8], w)
                for j in range(nj):
                    hj = plsc.load_gather(htab_v, [s16, jcols[j]])
                    plsc.store_scatter(buf, [rows, jcols[j]], w * hj)
                return carry

            lax.fori_loop(0, _NG, group_body, 0)

        def fire(p, k, b):
            pltpu.async_copy(ctrs[b], acc_sh.at[didxb[p].at[k]], sems[b],
                             add=True)

        def drain(b):
            pltpu.make_async_copy(ctrs[b], acc_sh.at[didxb[0].at[0]],
                                  sems[b]).wait()

        def sb_pair_body(ss, carry):
            for p in range(2):
                s = 2 * ss + p
                wait_idx(p)

                @pl.when(s < _NSB - 1)
                def _():
                    fire_idx(s + 1, 1 - p)

                for k in range(4):
                    b = k % 2

                    @pl.when(s * 4 + k >= 2)
                    def _():
                        drain(b)

                    compute_chunk(p, k, ctrs[b])
                    fire(p, k, b)
            return carry

        lax.fori_loop(0, _NSB // 2, sb_pair_body, 0)
        drain(0)
        drain(1)
        pltpu.sync_copy(src_hbm.at[pl.ds(cbase + _NCH - 1, 1)],
                        sidxb[0].at[pl.ds(0, 1)])
        pltpu.sync_copy(dst_hbm.at[pl.ds(cbase + _NCH - 1, 1)],
                        didxb[0].at[pl.ds(0, 1)])
        compute_chunk(0, 0, ctrs[0])
        fire(0, 0, 0)
        drain(0)
        plsc.subcore_barrier()
        pltpu.sync_copy(acc_sh.at[pl.ds(sid * _RPS, _RPS)],
                        acc_out.at[cid, pl.ds(sid * _RPS, _RPS)])

    return _sc_edges


_sc_edges_l1 = _make_sc_edges(8)
_sc_edges_l2 = _make_sc_edges(7)


def kernel(x, edge_index, W1, a_src1, a_dst1, b1, W2, a_src2, a_dst2, b2):
    zeros = jnp.zeros((_NNP, 16), jnp.float32)
    htab1, es1, ed1, c1, src3, dst3 = _prep1(
        x, edge_index.astype(jnp.int32), W1,
        a_src1.reshape(1, 8), a_dst1.reshape(1, 8))
    acc1 = _sc_edges_l1(src3, dst3, htab1, es1, ed1, c1, zeros)
    htab2, es2, ed2, c2 = _mid(acc1, b1.reshape(1, 8), W2,
                               a_src2.reshape(1, 7), a_dst2.reshape(1, 7))
    acc2 = _sc_edges_l2(src3, dst3, htab2, es2, ed2, c2, zeros)
    return _final(acc2, b2.reshape(1, 7))
